# baseline (speedup 1.0000x reference)
import jax
import jax.numpy as jnp
from jax.experimental import pallas as pl


def _gate_kernel(x_ref, z_ref, w1_ref, w2_ref, b_ref, out_ref):
    logits = jnp.dot(x_ref[...], w1_ref[...], preferred_element_type=jnp.float32)
    logits += jnp.dot(z_ref[...], w2_ref[...], preferred_element_type=jnp.float32)
    logits += b_ref[...]
    m = jnp.max(logits, axis=1, keepdims=True)
    e = jnp.exp(logits - m)
    out_ref[...] = e / jnp.sum(e, axis=1, keepdims=True)


def kernel(x, z, W, b):
    n_tokens, dx = x.shape
    dz = z.shape[1]
    num_experts = W.shape[1]
    w1 = W[:dx]
    w2 = W[dx:]
    b2 = b.reshape(1, num_experts)

    block = 512
    grid = (n_tokens // block,)

    return pl.pallas_call(
        _gate_kernel,
        grid=grid,
        in_specs=[
            pl.BlockSpec((block, dx), lambda i: (i, 0)),
            pl.BlockSpec((block, dz), lambda i: (i, 0)),
            pl.BlockSpec((dx, num_experts), lambda i: (0, 0)),
            pl.BlockSpec((dz, num_experts), lambda i: (0, 0)),
            pl.BlockSpec((1, num_experts), lambda i: (0, 0)),
        ],
        out_specs=pl.BlockSpec((block, num_experts), lambda i: (i, 0)),
        out_shape=jax.ShapeDtypeStruct((n_tokens, num_experts), jnp.float32),
    )(x, z, w1, w2, b2)
